# baseline (speedup 1.0000x reference)
import functools

import jax
import jax.numpy as jnp
from jax import lax
from jax.experimental import pallas as pl
from jax.experimental.pallas import tpu as pltpu
from jax.experimental.pallas import tpu_sc as plsc

N_NODES = 10000
N_EDGES = 320000
H = 128

NC = 2
NS = 16
NW = NC * NS
EPW = N_EDGES // NW
CHUNK = 80
NCHUNK = EPW // CHUNK
NBUF = 3
NGROUP = (NCHUNK + NBUF - 1) // NBUF
NPT = 632
NPT_LAST = N_NODES - 15 * NPT

_mesh = plsc.VectorSubcoreMesh(core_axis_name="c", subcore_axis_name="s")


@functools.partial(
    pl.kernel,
    mesh=_mesh,
    out_type=jax.ShapeDtypeStruct((NC, N_NODES, H), jnp.float32),
    scratch_types=[
        pltpu.VMEM((NCHUNK, CHUNK), jnp.int32),
        pltpu.VMEM((NBUF, CHUNK, H), jnp.float32),
    ] + [pltpu.SemaphoreType.DMA] * (2 * NBUF + 1) + [
        pltpu.VMEM_SHARED((N_NODES, H), jnp.float32),
    ],
)
def _sc_segment_sum(row_hbm, attr_hbm, out_hbm, idx_v, rows_v,
                    s0, s1, s2, t0, t1, t2, si, acc_sh):
    c = lax.axis_index("c")
    s = lax.axis_index("s")
    wid = c * NS + s
    sems = (s0, s1, s2)
    ssems = (t0, t1, t2)

    ebase = wid * EPW

    pltpu.async_copy(row_hbm.at[0, wid], idx_v, si)
    for b in range(1, NBUF):
        pltpu.async_copy(
            attr_hbm.at[pl.ds(ebase + b * CHUNK, CHUNK)], rows_v.at[b], sems[b])

    def zrow(r, carry):
        for j in range(H // 16):
            rows_v[0, r, pl.ds(j * 16, 16)] = jnp.zeros((16,), jnp.float32)
        return carry

    lax.fori_loop(0, CHUNK, zrow, 0)
    nbase = s * NPT

    @pl.when(s < NS - 1)
    def _():
        for j in range(7):
            pltpu.sync_copy(rows_v.at[0],
                            acc_sh.at[pl.ds(nbase + j * CHUNK, CHUNK)])
        pltpu.sync_copy(rows_v.at[0, pl.ds(0, NPT - 7 * CHUNK)],
                        acc_sh.at[pl.ds(nbase + 7 * CHUNK, NPT - 7 * CHUNK)])

    @pl.when(s == NS - 1)
    def _():
        for j in range(6):
            pltpu.sync_copy(rows_v.at[0],
                            acc_sh.at[pl.ds(15 * NPT + j * CHUNK, CHUNK)])
        pltpu.sync_copy(rows_v.at[0, pl.ds(0, NPT_LAST - 6 * CHUNK)],
                        acc_sh.at[pl.ds(15 * NPT + 6 * CHUNK,
                                        NPT_LAST - 6 * CHUNK)])

    pltpu.async_copy(attr_hbm.at[pl.ds(ebase, CHUNK)], rows_v.at[0], sems[0])
    pltpu.make_async_copy(row_hbm.at[0, wid], idx_v, si).wait()
    plsc.subcore_barrier()

    def group(g, carry):
        for b in range(NBUF):
            k = g * NBUF + b
            pb = (b + 1) % NBUF

            @pl.when(k < NCHUNK)
            def _():
                pltpu.make_async_copy(
                    attr_hbm.at[pl.ds(ebase + k * CHUNK, CHUNK)], rows_v.at[b],
                    sems[b]).wait()
                pltpu.async_copy(rows_v.at[b], acc_sh.at[idx_v.at[k]],
                                 ssems[b], add=True)

                @pl.when(jnp.logical_and(k >= NBUF - 1, k + 1 < NCHUNK))
                def _():
                    pltpu.make_async_copy(
                        rows_v.at[pb], acc_sh.at[idx_v.at[k - (NBUF - 1)]],
                        ssems[pb]).wait()
                    pltpu.async_copy(
                        attr_hbm.at[pl.ds(ebase + (k + 1) * CHUNK, CHUNK)],
                        rows_v.at[pb], sems[pb])
        return carry

    lax.fori_loop(0, NGROUP, group, 0)
    for k in (NCHUNK - 3, NCHUNK - 2, NCHUNK - 1):
        pltpu.make_async_copy(rows_v.at[k % NBUF], acc_sh.at[idx_v.at[k]],
                              ssems[k % NBUF]).wait()
    plsc.subcore_barrier()

    @pl.when(s < NS - 1)
    def _():
        pltpu.sync_copy(acc_sh.at[pl.ds(nbase, NPT)],
                        out_hbm.at[c, pl.ds(nbase, NPT)])

    @pl.when(s == NS - 1)
    def _():
        pltpu.sync_copy(acc_sh.at[pl.ds(15 * NPT, NPT_LAST)],
                        out_hbm.at[c, pl.ds(15 * NPT, NPT_LAST)])


ROWS_BLK = 2000


def _mlp_body(x_ref, agg_ref, w1a_ref, w1b_ref, b1_ref, w2_ref, b2_ref,
              out_ref, comb_ref):
    xb = x_ref[...]
    ab = agg_ref[0] + agg_ref[1]
    comb_ref[:, :H] = xb
    comb_ref[:, H:] = ab
    h = jnp.dot(xb, w1a_ref[...], preferred_element_type=jnp.float32)
    h += jnp.dot(ab, w1b_ref[...], preferred_element_type=jnp.float32)
    h = jnp.maximum(h + b1_ref[...], 0.0)
    out_ref[...] = jnp.dot(h, w2_ref[...], preferred_element_type=jnp.float32) + b2_ref[...]


def _mlp(x, agg2, W1, b1, W2, b2):
    grid = (N_NODES // ROWS_BLK,)
    out, comb = pl.pallas_call(
        _mlp_body,
        grid=grid,
        in_specs=[
            pl.BlockSpec((ROWS_BLK, H), lambda i: (i, 0)),
            pl.BlockSpec((NC, ROWS_BLK, H), lambda i: (0, i, 0)),
            pl.BlockSpec((H, H), lambda i: (0, 0)),
            pl.BlockSpec((H, H), lambda i: (0, 0)),
            pl.BlockSpec((1, H), lambda i: (0, 0)),
            pl.BlockSpec((H, H), lambda i: (0, 0)),
            pl.BlockSpec((1, H), lambda i: (0, 0)),
        ],
        out_specs=[
            pl.BlockSpec((ROWS_BLK, H), lambda i: (i, 0)),
            pl.BlockSpec((ROWS_BLK, 2 * H), lambda i: (i, 0)),
        ],
        out_shape=[
            jax.ShapeDtypeStruct((N_NODES, H), jnp.float32),
            jax.ShapeDtypeStruct((N_NODES, 2 * H), jnp.float32),
        ],
    )(x, agg2, W1[:H], W1[H:], b1.reshape(1, H), W2, b2.reshape(1, H))
    return out, comb


def kernel(edge_index, edge_attr, x, W1, b1, W2, b2):
    ei = edge_index.astype(jnp.int32).reshape(2, NW, NCHUNK, CHUNK)
    agg2 = _sc_segment_sum(ei, edge_attr)
    return _mlp(x, agg2, W1, b1, W2, b2)


# speedup vs baseline: 1.4551x; 1.4551x over previous
import functools

import jax
import jax.numpy as jnp
from jax import lax
from jax.experimental import pallas as pl
from jax.experimental.pallas import tpu as pltpu
from jax.experimental.pallas import tpu_sc as plsc

N_NODES = 10000
N_EDGES = 320000
H = 128

NC = 2
NS = 16
NW = NC * NS
EPW = N_EDGES // NW
CHUNK = 80
NCHUNK = EPW // CHUNK
NBUF = 3
NGROUP = (NCHUNK + NBUF - 1) // NBUF
NPT = 632
NPT_LAST = N_NODES - 15 * NPT

_mesh = plsc.VectorSubcoreMesh(core_axis_name="c", subcore_axis_name="s")


@functools.partial(
    pl.kernel,
    mesh=_mesh,
    out_type=jax.ShapeDtypeStruct((NC, N_NODES, H), jnp.float32),
    scratch_types=[
        pltpu.VMEM((NCHUNK, CHUNK), jnp.int32),
        pltpu.VMEM((NBUF, CHUNK, H), jnp.float32),
    ] + [pltpu.SemaphoreType.DMA] * (NBUF + 1) + [
        pltpu.VMEM_SHARED((N_NODES, H), jnp.float32),
    ],
)
def _sc_segment_sum(row_hbm, attr_hbm, out_hbm, idx_v, rows_v,
                    s0, s1, s2, si, acc_sh):
    c = lax.axis_index("c")
    s = lax.axis_index("s")
    wid = c * NS + s
    sems = (s0, s1, s2)

    ebase = wid * EPW

    pltpu.async_copy(row_hbm.at[0, wid], idx_v, si)
    for b in range(1, NBUF):
        pltpu.async_copy(
            attr_hbm.at[pl.ds(ebase + b * CHUNK, CHUNK)], rows_v.at[b], sems[b])

    def zrow(r, carry):
        for j in range(H // 16):
            rows_v[0, r, pl.ds(j * 16, 16)] = jnp.zeros((16,), jnp.float32)
        return carry

    lax.fori_loop(0, CHUNK, zrow, 0)
    nbase = s * NPT

    @pl.when(s < NS - 1)
    def _():
        for j in range(7):
            pltpu.sync_copy(rows_v.at[0],
                            acc_sh.at[pl.ds(nbase + j * CHUNK, CHUNK)])
        pltpu.sync_copy(rows_v.at[0, pl.ds(0, NPT - 7 * CHUNK)],
                        acc_sh.at[pl.ds(nbase + 7 * CHUNK, NPT - 7 * CHUNK)])

    @pl.when(s == NS - 1)
    def _():
        for j in range(6):
            pltpu.sync_copy(rows_v.at[0],
                            acc_sh.at[pl.ds(15 * NPT + j * CHUNK, CHUNK)])
        pltpu.sync_copy(rows_v.at[0, pl.ds(0, NPT_LAST - 6 * CHUNK)],
                        acc_sh.at[pl.ds(15 * NPT + 6 * CHUNK,
                                        NPT_LAST - 6 * CHUNK)])

    pltpu.async_copy(attr_hbm.at[pl.ds(ebase, CHUNK)], rows_v.at[0], sems[0])
    pltpu.make_async_copy(row_hbm.at[0, wid], idx_v, si).wait()
    plsc.subcore_barrier()

    def group(g, carry):
        for b in range(NBUF):
            k = g * NBUF + b

            @pl.when(k < NCHUNK)
            def _():
                pltpu.make_async_copy(
                    attr_hbm.at[pl.ds(ebase + k * CHUNK, CHUNK)], rows_v.at[b],
                    sems[b]).wait()
                pltpu.sync_copy(rows_v.at[b], acc_sh.at[idx_v.at[k]], add=True)
                nk = k + NBUF

                @pl.when(nk < NCHUNK)
                def _():
                    pltpu.async_copy(
                        attr_hbm.at[pl.ds(ebase + nk * CHUNK, CHUNK)],
                        rows_v.at[b], sems[b])
        return carry

    lax.fori_loop(0, NGROUP, group, 0)
    plsc.subcore_barrier()

    @pl.when(s < NS - 1)
    def _():
        pltpu.sync_copy(acc_sh.at[pl.ds(nbase, NPT)],
                        out_hbm.at[c, pl.ds(nbase, NPT)])

    @pl.when(s == NS - 1)
    def _():
        pltpu.sync_copy(acc_sh.at[pl.ds(15 * NPT, NPT_LAST)],
                        out_hbm.at[c, pl.ds(15 * NPT, NPT_LAST)])


ROWS_BLK = 2000


def _mlp_body(x_ref, agg_ref, w1a_ref, w1b_ref, b1_ref, w2_ref, b2_ref,
              out_ref, comb_ref):
    xb = x_ref[...]
    ab = agg_ref[0] + agg_ref[1]
    comb_ref[:, :H] = xb
    comb_ref[:, H:] = ab
    h = jnp.dot(xb, w1a_ref[...], preferred_element_type=jnp.float32)
    h += jnp.dot(ab, w1b_ref[...], preferred_element_type=jnp.float32)
    h = jnp.maximum(h + b1_ref[...], 0.0)
    out_ref[...] = jnp.dot(h, w2_ref[...], preferred_element_type=jnp.float32) + b2_ref[...]


def _mlp(x, agg2, W1, b1, W2, b2):
    grid = (N_NODES // ROWS_BLK,)
    out, comb = pl.pallas_call(
        _mlp_body,
        grid=grid,
        in_specs=[
            pl.BlockSpec((ROWS_BLK, H), lambda i: (i, 0)),
            pl.BlockSpec((NC, ROWS_BLK, H), lambda i: (0, i, 0)),
            pl.BlockSpec((H, H), lambda i: (0, 0)),
            pl.BlockSpec((H, H), lambda i: (0, 0)),
            pl.BlockSpec((1, H), lambda i: (0, 0)),
            pl.BlockSpec((H, H), lambda i: (0, 0)),
            pl.BlockSpec((1, H), lambda i: (0, 0)),
        ],
        out_specs=[
            pl.BlockSpec((ROWS_BLK, H), lambda i: (i, 0)),
            pl.BlockSpec((ROWS_BLK, 2 * H), lambda i: (i, 0)),
        ],
        out_shape=[
            jax.ShapeDtypeStruct((N_NODES, H), jnp.float32),
            jax.ShapeDtypeStruct((N_NODES, 2 * H), jnp.float32),
        ],
    )(x, agg2, W1[:H], W1[H:], b1.reshape(1, H), W2, b2.reshape(1, H))
    return out, comb


def kernel(edge_index, edge_attr, x, W1, b1, W2, b2):
    ei = edge_index.astype(jnp.int32).reshape(2, NW, NCHUNK, CHUNK)
    agg2 = _sc_segment_sum(ei, edge_attr)
    return _mlp(x, agg2, W1, b1, W2, b2)
